# baseline (speedup 1.0000x reference)
import functools
import math

import jax
import jax.numpy as jnp
from jax.experimental import pallas as pl
from jax.experimental.pallas import tpu as pltpu

H = 12

_C00 = (((0,), (0,)), ((), ()))
_C11 = (((1,), (1,)), ((), ()))
_C10 = (((1,), (0,)), ((), ()))


def _qkv_proj_kernel(xq_ref, xk_ref, xv_ref, w_ref, b_ref, qo_ref, ko_ref, vo_ref,
                     *, n_b, e, scale):
    for b in range(n_b):
        xq = xq_ref[:, b, :]
        xk = xk_ref[:, b, :]
        xv = xv_ref[:, b, :]
        rows = slice(b * e, (b + 1) * e)
        qo_ref[rows, :] = ((jax.lax.dot_general(
            w_ref[:e, :], xq, _C11, preferred_element_type=jnp.float32)
            + b_ref[:e, :]) * scale).astype(jnp.bfloat16)
        ko_ref[rows, :] = (jax.lax.dot_general(
            w_ref[e:2 * e, :], xk, _C11, preferred_element_type=jnp.float32
        ) + b_ref[e:2 * e, :]).astype(jnp.bfloat16)
        vo_ref[rows, :] = (jax.lax.dot_general(
            w_ref[2 * e:, :], xv, _C11, preferred_element_type=jnp.float32
        ) + b_ref[2 * e:, :]).astype(jnp.bfloat16)


def _head_chain(q_ref, k_ref, v_ref, wot_ref):
    s = jax.lax.dot_general(q_ref[...], k_ref[...], _C00,
                            preferred_element_type=jnp.float32)
    m = jnp.max(s, axis=-1, keepdims=True)
    ex = jnp.exp2(s - m).astype(jnp.bfloat16)
    r = 1.0 / jnp.sum(ex, axis=-1, keepdims=True, dtype=jnp.float32)
    pv = jax.lax.dot_general(ex, v_ref[...], _C11,
                             preferred_element_type=jnp.float32) * r
    o = jax.lax.dot_general(pv, wot_ref[0], _C10,
                            preferred_element_type=jnp.float32)
    return ex, r, o


def _attn_kernel(*refs, tq, n_ch):
    in_refs, (ob_ref, out_ref, aw_ref) = refs[:-3], refs[-3:]
    hp = pl.program_id(2)
    chains = [_head_chain(*in_refs[4 * c:4 * c + 4]) for c in range(n_ch)]
    first = hp == 0
    base_o = jnp.where(first, 0.0, out_ref[...])
    bias = jnp.where(first, ob_ref[0], 0.0)
    o_sum = chains[0][2]
    for c in range(1, n_ch):
        o_sum = o_sum + chains[c][2]
    out_ref[...] = base_o + (o_sum + bias)
    base_aw = jnp.where(first, 0.0, aw_ref[0])
    aw_sum = chains[0][0] * (chains[0][1] * (1.0 / H))
    for c in range(1, n_ch):
        aw_sum = aw_sum + chains[c][0] * (chains[c][1] * (1.0 / H))
    aw_ref[0] = base_aw + aw_sum


def kernel(query, key, value, in_proj_weight, in_proj_bias, out_proj_w, out_proj_b,
           dgl_ln_g, dgl_ln_b, dgl_w1, dgl_b1, dgl_w2, dgl_b2):
    T, B, E = query.shape
    S = key.shape[0]
    hd = E // H
    scale = math.log2(math.e) / math.sqrt(hd)

    tt = 512
    qt, kt, vt = pl.pallas_call(
        functools.partial(_qkv_proj_kernel, n_b=B, e=E, scale=scale),
        grid=(T // tt,),
        in_specs=[
            pl.BlockSpec((tt, B, E), lambda i: (i, 0, 0)),
            pl.BlockSpec((tt, B, E), lambda i: (i, 0, 0)),
            pl.BlockSpec((tt, B, E), lambda i: (i, 0, 0)),
            pl.BlockSpec((3 * E, E), lambda i: (0, 0)),
            pl.BlockSpec((3 * E, 1), lambda i: (0, 0)),
        ],
        out_specs=[pl.BlockSpec((B * E, tt), lambda i: (0, i))] * 3,
        out_shape=[jax.ShapeDtypeStruct((B * E, T), jnp.bfloat16)] * 3,
        compiler_params=pltpu.CompilerParams(
            dimension_semantics=("parallel",)),
    )(query, key, value, in_proj_weight, in_proj_bias.reshape(3 * E, 1))

    wot = out_proj_w.T.reshape(H, hd, E)

    tq = 1024
    n_ch = 2
    hh = H // n_ch
    in_specs = []
    args = []
    for c in range(n_ch):
        base = c * hh
        in_specs += [
            pl.BlockSpec((hd, tq),
                         (lambda bs: lambda b, i, h: (b * H + bs + h, i))(base)),
            pl.BlockSpec((hd, S),
                         (lambda bs: lambda b, i, h: (b * H + bs + h, 0))(base)),
            pl.BlockSpec((hd, S),
                         (lambda bs: lambda b, i, h: (b * H + bs + h, 0))(base)),
            pl.BlockSpec((1, hd, E),
                         (lambda bs: lambda b, i, h: (bs + h, 0, 0))(base)),
        ]
        args += [qt, kt, vt, wot]
    in_specs.append(pl.BlockSpec((1, E), lambda b, i, h: (0, 0)))
    args.append(out_proj_b.reshape(1, E))
    out, aw = pl.pallas_call(
        functools.partial(_attn_kernel, tq=tq, n_ch=n_ch),
        grid=(B, T // tq, hh),
        in_specs=in_specs,
        out_specs=[
            pl.BlockSpec((tq, E), lambda b, i, h: (i, b)),
            pl.BlockSpec((1, tq, S), lambda b, i, h: (b, i, 0)),
        ],
        out_shape=[
            jax.ShapeDtypeStruct((T, B * E), jnp.float32),
            jax.ShapeDtypeStruct((B, T, S), jnp.float32),
        ],
        compiler_params=pltpu.CompilerParams(
            dimension_semantics=("parallel", "arbitrary", "arbitrary")),
    )(*args)

    return out.reshape(T, B, E), aw


# speedup vs baseline: 1.0497x; 1.0497x over previous
import functools
import math

import jax
import jax.numpy as jnp
from jax.experimental import pallas as pl
from jax.experimental.pallas import tpu as pltpu

H = 12

_C00 = (((0,), (0,)), ((), ()))
_C11 = (((1,), (1,)), ((), ()))
_C10 = (((1,), (0,)), ((), ()))


def _qkv_proj_kernel(xq_ref, xk_ref, xv_ref, w_ref, b_ref, qo_ref, ko_ref, vo_ref,
                     *, n_b, e, scale):
    for b in range(n_b):
        xq = xq_ref[:, b, :]
        xk = xk_ref[:, b, :]
        xv = xv_ref[:, b, :]
        rows = slice(b * e, (b + 1) * e)
        qo_ref[rows, :] = ((jax.lax.dot_general(
            w_ref[:e, :], xq, _C11, preferred_element_type=jnp.float32)
            + b_ref[:e, :]) * scale).astype(jnp.bfloat16)
        ko_ref[rows, :] = (jax.lax.dot_general(
            w_ref[e:2 * e, :], xk, _C11, preferred_element_type=jnp.float32
        ) + b_ref[e:2 * e, :]).astype(jnp.bfloat16)
        vo_ref[rows, :] = (jax.lax.dot_general(
            w_ref[2 * e:, :], xv, _C11, preferred_element_type=jnp.float32
        ) + b_ref[2 * e:, :]).astype(jnp.bfloat16)


def _head_chain(q_ref, k_ref, v_ref, wot_ref):
    s = jax.lax.dot_general(q_ref[...], k_ref[...], _C00,
                            preferred_element_type=jnp.float32)
    m = jnp.max(s, axis=-1, keepdims=True)
    ex = jnp.exp2(s - m)
    r = 1.0 / jnp.sum(ex, axis=-1, keepdims=True)
    pv = jax.lax.dot_general(ex.astype(jnp.bfloat16), v_ref[...], _C11,
                             preferred_element_type=jnp.float32) * r
    o = jax.lax.dot_general(pv, wot_ref[0], _C10,
                            preferred_element_type=jnp.float32)
    return ex, r, o


def _attn_kernel(*refs, tq, n_ch):
    in_refs, (ob_ref, out_ref, aw_ref) = refs[:-3], refs[-3:]
    hp = pl.program_id(2)
    chains = [_head_chain(*in_refs[4 * c:4 * c + 4]) for c in range(n_ch)]
    first = hp == 0
    base_o = jnp.where(first, 0.0, out_ref[...])
    bias = jnp.where(first, ob_ref[0], 0.0)
    o_sum = chains[0][2]
    for c in range(1, n_ch):
        o_sum = o_sum + chains[c][2]
    out_ref[...] = base_o + (o_sum + bias)
    base_aw = jnp.where(first, 0.0, aw_ref[0])
    aw_sum = chains[0][0] * (chains[0][1] * (1.0 / H))
    for c in range(1, n_ch):
        aw_sum = aw_sum + chains[c][0] * (chains[c][1] * (1.0 / H))
    aw_ref[0] = base_aw + aw_sum


def kernel(query, key, value, in_proj_weight, in_proj_bias, out_proj_w, out_proj_b,
           dgl_ln_g, dgl_ln_b, dgl_w1, dgl_b1, dgl_w2, dgl_b2):
    T, B, E = query.shape
    S = key.shape[0]
    hd = E // H
    scale = math.log2(math.e) / math.sqrt(hd)

    tt = 512
    qt, kt, vt = pl.pallas_call(
        functools.partial(_qkv_proj_kernel, n_b=B, e=E, scale=scale),
        grid=(T // tt,),
        in_specs=[
            pl.BlockSpec((tt, B, E), lambda i: (i, 0, 0)),
            pl.BlockSpec((tt, B, E), lambda i: (i, 0, 0)),
            pl.BlockSpec((tt, B, E), lambda i: (i, 0, 0)),
            pl.BlockSpec((3 * E, E), lambda i: (0, 0)),
            pl.BlockSpec((3 * E, 1), lambda i: (0, 0)),
        ],
        out_specs=[pl.BlockSpec((B * E, tt), lambda i: (0, i))] * 3,
        out_shape=[jax.ShapeDtypeStruct((B * E, T), jnp.bfloat16)] * 3,
        compiler_params=pltpu.CompilerParams(
            dimension_semantics=("parallel",)),
    )(query, key, value, in_proj_weight, in_proj_bias.reshape(3 * E, 1))

    wot = out_proj_w.T.reshape(H, hd, E)

    tq = 1024
    n_ch = 2
    hh = H // n_ch
    in_specs = []
    args = []
    for c in range(n_ch):
        base = c * hh
        in_specs += [
            pl.BlockSpec((hd, tq),
                         (lambda bs: lambda b, i, h: (b * H + bs + h, i))(base)),
            pl.BlockSpec((hd, S),
                         (lambda bs: lambda b, i, h: (b * H + bs + h, 0))(base)),
            pl.BlockSpec((hd, S),
                         (lambda bs: lambda b, i, h: (b * H + bs + h, 0))(base)),
            pl.BlockSpec((1, hd, E),
                         (lambda bs: lambda b, i, h: (bs + h, 0, 0))(base)),
        ]
        args += [qt, kt, vt, wot]
    in_specs.append(pl.BlockSpec((1, E), lambda b, i, h: (0, 0)))
    args.append(out_proj_b.reshape(1, E))
    out, aw = pl.pallas_call(
        functools.partial(_attn_kernel, tq=tq, n_ch=n_ch),
        grid=(B, T // tq, hh),
        in_specs=in_specs,
        out_specs=[
            pl.BlockSpec((tq, E), lambda b, i, h: (i, b)),
            pl.BlockSpec((1, tq, S), lambda b, i, h: (b, i, 0)),
        ],
        out_shape=[
            jax.ShapeDtypeStruct((T, B * E), jnp.float32),
            jax.ShapeDtypeStruct((B, T, S), jnp.float32),
        ],
        compiler_params=pltpu.CompilerParams(
            dimension_semantics=("parallel", "parallel", "arbitrary")),
    )(*args)

    return out.reshape(T, B, E), aw
